# baseline (speedup 1.0000x reference)
import functools

import jax
import jax.numpy as jnp
from jax import lax
from jax.experimental import pallas as pl
from jax.experimental.pallas import tpu as pltpu
from jax.experimental.pallas import tpu_sc as plsc

SEQ = 8192
D_EMB = 768
NUM_CORES = 2
NUM_SUBCORES = 16
LANES = 16
NW = NUM_CORES * NUM_SUBCORES
ROWS_PER_W = SEQ // NW
CHUNK = 64
NCHUNK = ROWS_PER_W // CHUNK

_mesh = plsc.VectorSubcoreMesh(core_axis_name="c", subcore_axis_name="s")


@functools.partial(
    pl.kernel,
    mesh=_mesh,
    out_type=jax.ShapeDtypeStruct((SEQ, D_EMB), jnp.float32),
    scratch_types=[
        pltpu.VMEM((ROWS_PER_W,), jnp.int32),
        pltpu.VMEM((CHUNK, D_EMB), jnp.float32),
        pltpu.VMEM((CHUNK, D_EMB), jnp.float32),
        pltpu.SemaphoreType.DMA,
    ],
)
def _embed(emb_hbm, pos_hbm, idx_hbm, out_hbm, idx_v, tok_v, pos_v, sem):
    wid = lax.axis_index("s") * NUM_CORES + lax.axis_index("c")
    base = wid * ROWS_PER_W
    pltpu.sync_copy(idx_hbm.at[pl.ds(base, ROWS_PER_W)], idx_v)
    for ci in range(NCHUNK):
        cbase = ci * CHUNK
        cp = pltpu.async_copy(
            emb_hbm.at[idx_v.at[pl.ds(cbase, CHUNK)]], tok_v, sem)
        pltpu.sync_copy(pos_hbm.at[pl.ds(base + cbase, CHUNK)], pos_v)
        cp.wait()

        def row_body(r, _):
            def col_body(c, _):
                s = pl.ds(c * LANES, LANES)
                pos_v[r, s] = pos_v[r, s] + tok_v[r, s]
                return 0
            return lax.fori_loop(0, D_EMB // LANES, col_body, 0)

        lax.fori_loop(0, CHUNK, row_body, 0)
        pltpu.sync_copy(pos_v, out_hbm.at[pl.ds(base + cbase, CHUNK)])


def kernel(x, embedding, positional):
    return _embed(embedding, positional, x)


# speedup vs baseline: 1.9679x; 1.9679x over previous
import functools

import jax
import jax.numpy as jnp
from jax import lax
from jax.experimental import pallas as pl
from jax.experimental.pallas import tpu as pltpu
from jax.experimental.pallas import tpu_sc as plsc

SEQ = 8192
D_EMB = 768
NUM_CORES = 2
NUM_SUBCORES = 16
LANES = 16
NW = NUM_CORES * NUM_SUBCORES
ROWS_PER_W = SEQ // NW
CHUNK = 32
NCHUNK = ROWS_PER_W // CHUNK
NCOL = D_EMB // LANES

_mesh = plsc.VectorSubcoreMesh(core_axis_name="c", subcore_axis_name="s")


@functools.partial(
    pl.kernel,
    mesh=_mesh,
    out_type=jax.ShapeDtypeStruct((SEQ, D_EMB), jnp.float32),
    scratch_types=[
        pltpu.VMEM((ROWS_PER_W,), jnp.int32),
        pltpu.VMEM((CHUNK, D_EMB), jnp.float32),
        pltpu.VMEM((CHUNK, D_EMB), jnp.float32),
        pltpu.VMEM((CHUNK, D_EMB), jnp.float32),
        pltpu.VMEM((CHUNK, D_EMB), jnp.float32),
        pltpu.SemaphoreType.DMA,
        pltpu.SemaphoreType.DMA,
        pltpu.SemaphoreType.DMA,
        pltpu.SemaphoreType.DMA,
        pltpu.SemaphoreType.DMA,
        pltpu.SemaphoreType.DMA,
    ],
)
def _embed(emb_hbm, pos_hbm, idx_hbm, out_hbm,
           idx_v, tok0, tok1, pos0, pos1,
           sg0, sg1, sp0, sp1, so0, so1):
    tok = (tok0, tok1)
    pos = (pos0, pos1)
    sg = (sg0, sg1)
    sp = (sp0, sp1)
    so = (so0, so1)

    wid = lax.axis_index("s") * NUM_CORES + lax.axis_index("c")
    base = wid * ROWS_PER_W
    pltpu.sync_copy(idx_hbm.at[pl.ds(base, ROWS_PER_W)], idx_v)

    def issue(ci, b):
        cbase = ci * CHUNK
        g = pltpu.async_copy(
            emb_hbm.at[idx_v.at[pl.ds(cbase, CHUNK)]], tok[b], sg[b])
        p = pltpu.async_copy(
            pos_hbm.at[pl.ds(base + cbase, CHUNK)], pos[b], sp[b])
        return g, p

    pend = [None, None]
    inflight = issue(0, 0)
    for ci in range(NCHUNK):
        b = ci % 2
        nxt = None
        if ci + 1 < NCHUNK:
            nxt = issue(ci + 1, 1 - b)
        g, p = inflight
        g.wait()
        p.wait()
        if pend[b] is not None:
            pend[b].wait()
            pend[b] = None

        def row_body(r, _):
            for c in range(NCOL):
                s = pl.ds(c * LANES, LANES)
                plsc.addupdate(pos[b].at[r, s], tok[b][r, s])
            return 0

        lax.fori_loop(0, CHUNK, row_body, 0, unroll=2)
        pend[b] = pltpu.async_copy(
            pos[b], out_hbm.at[pl.ds(base + ci * CHUNK, CHUNK)], so[b])
        inflight = nxt
    for w in pend:
        if w is not None:
            w.wait()


def kernel(x, embedding, positional):
    return _embed(embedding, positional, x)
